# Initial kernel scaffold; baseline (speedup 1.0000x reference)
import jax
import jax.numpy as jnp
from jax.experimental import pallas as pl


def kernel(x):
    raise NotImplementedError("write your pallas kernel here")


# speedup vs baseline: 4.7423x; 4.7423x over previous
import jax
import jax.numpy as jnp
from jax.experimental import pallas as pl
from jax.experimental.pallas import tpu as pltpu

_MASK_RATIO = 0.5
_MASK_VALUE = 0.0
_BLK = 16


def _threefry_sortkeys(n):
    rotations = ((13, 15, 26, 6), (17, 29, 16, 24))
    k = (jnp.uint32(0), jnp.uint32(42),
         jnp.uint32(0) ^ jnp.uint32(42) ^ jnp.uint32(0x1BD11BDA))
    x0 = jnp.zeros_like(n) + k[0]
    x1 = n + k[1]
    for i in range(5):
        for r in rotations[i % 2]:
            x0 = x0 + x1
            x1 = (x1 << jnp.uint32(r)) | (x1 >> jnp.uint32(32 - r))
            x1 = x0 ^ x1
        x0 = x0 + k[(i + 1) % 3]
        x1 = x1 + k[(i + 2) % 3] + jnp.uint32(i + 1)
    bits = x0 ^ x1
    return (bits >> jnp.uint32(9)).astype(jnp.int32)


def _mask_body(x_ref, xm_ref, mask_ref):
    blk, nv, row = x_ref.shape
    L = mask_ref.shape[-1]
    D = row // L
    len_keep = int(L * (1 - _MASK_RATIO))
    b0 = pl.program_id(0) * blk

    bi = jax.lax.broadcasted_iota(jnp.int32, (blk, L), 0)
    li = jax.lax.broadcasted_iota(jnp.int32, (blk, L), 1)
    keys = _threefry_sortkeys(((b0 + bi) * L + li).astype(jnp.uint32))

    at_l = keys[:, :, None]
    at_j = keys[:, None, :]
    l3 = jax.lax.broadcasted_iota(jnp.int32, (blk, L, L), 1)
    j3 = jax.lax.broadcasted_iota(jnp.int32, (blk, L, L), 2)
    before = (at_j < at_l) | ((at_j == at_l) & (j3 < l3))
    rank = jnp.sum(before.astype(jnp.int32), axis=-1)
    mask = (rank >= len_keep).astype(jnp.float32)

    ci = jax.lax.broadcasted_iota(jnp.int32, (L, row), 1)
    ri = jax.lax.broadcasted_iota(jnp.int32, (L, row), 0)
    expand = (ci // D == ri).astype(jnp.bfloat16)
    mask_row = jax.lax.dot_general(
        mask.astype(jnp.bfloat16), expand, (((1,), (0,)), ((), ())),
        preferred_element_type=jnp.float32)

    xm_ref[...] = jnp.where(mask_row[:, None, :] == 0.0, x_ref[...],
                            jnp.float32(_MASK_VALUE))
    mask_ref[...] = jnp.broadcast_to(mask[:, None, :], (blk, nv, L))


def kernel(x):
    bs, nv, L, D = x.shape
    row = L * D
    x2 = x.reshape(bs, nv, row)
    grid = bs // _BLK
    xm2, mask = pl.pallas_call(
        _mask_body,
        grid=(grid,),
        in_specs=[pl.BlockSpec((_BLK, nv, row), lambda i: (i, 0, 0))],
        out_specs=[pl.BlockSpec((_BLK, nv, row), lambda i: (i, 0, 0)),
                   pl.BlockSpec((_BLK, nv, L), lambda i: (i, 0, 0))],
        out_shape=[jax.ShapeDtypeStruct((bs, nv, row), jnp.float32),
                   jax.ShapeDtypeStruct((bs, nv, L), jnp.float32)],
        compiler_params=pltpu.CompilerParams(
            dimension_semantics=("arbitrary",)),
    )(x2)
    return xm2.reshape(x.shape), mask


# speedup vs baseline: 19.1390x; 4.0358x over previous
import jax
import jax.numpy as jnp
from jax.experimental import pallas as pl
from jax.experimental.pallas import tpu as pltpu

_MASK_RATIO = 0.5
_MASK_VALUE = 0.0


def _threefry_sortkeys(n):
    rotations = ((13, 15, 26, 6), (17, 29, 16, 24))
    k = (jnp.uint32(0), jnp.uint32(42),
         jnp.uint32(0) ^ jnp.uint32(42) ^ jnp.uint32(0x1BD11BDA))
    x0 = jnp.zeros_like(n) + k[0]
    x1 = n + k[1]
    for i in range(5):
        for r in rotations[i % 2]:
            x0 = x0 + x1
            x1 = (x1 << jnp.uint32(r)) | (x1 >> jnp.uint32(32 - r))
            x1 = x0 ^ x1
        x0 = x0 + k[(i + 1) % 3]
        x1 = x1 + k[(i + 2) % 3] + jnp.uint32(i + 1)
    bits = x0 ^ x1
    return (bits >> jnp.uint32(9)).astype(jnp.int32)


def _mask_body(x_ref, xm_ref, mask_ref, keys_scr, keep_scr, maskv_scr):
    _, L, D, bs = x_ref.shape
    len_keep = int(L * (1 - _MASK_RATIO))

    @pl.when(pl.program_id(0) == 0)
    def _init():
        li = jax.lax.broadcasted_iota(jnp.int32, (L, bs), 0)
        bi = jax.lax.broadcasted_iota(jnp.int32, (L, bs), 1)
        keys = _threefry_sortkeys((bi * L + li).astype(jnp.uint32))
        keys_scr[...] = keys

        def body(j, rank):
            kj = keys_scr[pl.ds(j, 1), :]
            return rank + ((kj < keys) |
                           ((kj == keys) & (li > j))).astype(jnp.int32)

        rank = jax.lax.fori_loop(0, L, body, jnp.zeros((L, bs), jnp.int32))
        maskv = (rank >= len_keep).astype(jnp.float32)
        maskv_scr[...] = maskv
        keep_scr[...] = 1.0 - maskv

    xm_ref[...] = x_ref[...] * keep_scr[...][None, :, None, :]
    mask_ref[...] = maskv_scr[...][None, :, :]


def kernel(x):
    bs, nv, L, D = x.shape
    xt = jnp.transpose(x, (1, 2, 3, 0))
    xm_t, mask_t = pl.pallas_call(
        _mask_body,
        grid=(nv,),
        in_specs=[pl.BlockSpec((1, L, D, bs), lambda i: (i, 0, 0, 0))],
        out_specs=[pl.BlockSpec((1, L, D, bs), lambda i: (i, 0, 0, 0)),
                   pl.BlockSpec((1, L, bs), lambda i: (i, 0, 0))],
        out_shape=[jax.ShapeDtypeStruct((nv, L, D, bs), jnp.float32),
                   jax.ShapeDtypeStruct((nv, L, bs), jnp.float32)],
        scratch_shapes=[pltpu.VMEM((L, bs), jnp.int32),
                        pltpu.VMEM((L, bs), jnp.float32),
                        pltpu.VMEM((L, bs), jnp.float32)],
        compiler_params=pltpu.CompilerParams(
            dimension_semantics=("arbitrary",)),
    )(xt)
    return jnp.transpose(xm_t, (3, 0, 1, 2)), jnp.transpose(mask_t, (2, 0, 1))


# speedup vs baseline: 19.5914x; 1.0236x over previous
import jax
import jax.numpy as jnp
from jax.experimental import pallas as pl
from jax.experimental.pallas import tpu as pltpu

_MASK_RATIO = 0.5
_MASK_VALUE = 0.0


def _threefry_sortkeys(n):
    rotations = ((13, 15, 26, 6), (17, 29, 16, 24))
    k = (jnp.uint32(0), jnp.uint32(42),
         jnp.uint32(0) ^ jnp.uint32(42) ^ jnp.uint32(0x1BD11BDA))
    x0 = jnp.zeros_like(n) + k[0]
    x1 = n + k[1]
    for i in range(5):
        for r in rotations[i % 2]:
            x0 = x0 + x1
            x1 = (x1 << jnp.uint32(r)) | (x1 >> jnp.uint32(32 - r))
            x1 = x0 ^ x1
        x0 = x0 + k[(i + 1) % 3]
        x1 = x1 + k[(i + 2) % 3] + jnp.uint32(i + 1)
    bits = x0 ^ x1
    return (bits >> jnp.uint32(9)).astype(jnp.int32)


def _mask_body(x_ref, xm_ref, mask_ref, keys_scr, keep_scr, maskv_scr):
    _, L, D, bs = x_ref.shape
    len_keep = int(L * (1 - _MASK_RATIO))

    @pl.when(pl.program_id(0) == 0)
    def _init():
        li = jax.lax.broadcasted_iota(jnp.int32, (L, bs), 0)
        bi = jax.lax.broadcasted_iota(jnp.int32, (L, bs), 1)
        keys = _threefry_sortkeys((bi * L + li).astype(jnp.uint32))
        keys_scr[...] = keys

        def body(j, rank):
            kj = keys_scr[pl.ds(j, 1), :]
            return rank + ((kj < keys) |
                           ((kj == keys) & (li > j))).astype(jnp.int32)

        rank = jax.lax.fori_loop(0, L, body, jnp.zeros((L, bs), jnp.int32))
        maskv = (rank >= len_keep).astype(jnp.float32)
        maskv_scr[...] = maskv
        keep_scr[...] = 1.0 - maskv

    xm_ref[...] = x_ref[...] * keep_scr[...][None, :, None, :]
    mask_ref[...] = jnp.broadcast_to(maskv_scr[...][None, :, :],
                                     mask_ref.shape)


def kernel(x):
    bs, nv, L, D = x.shape
    xt = jnp.transpose(x, (1, 2, 3, 0))
    V = 2
    xm_t, mask_t = pl.pallas_call(
        _mask_body,
        grid=(nv // V,),
        in_specs=[pl.BlockSpec((V, L, D, bs), lambda i: (i, 0, 0, 0))],
        out_specs=[pl.BlockSpec((V, L, D, bs), lambda i: (i, 0, 0, 0)),
                   pl.BlockSpec((V, L, bs), lambda i: (i, 0, 0))],
        out_shape=[jax.ShapeDtypeStruct((nv, L, D, bs), jnp.float32),
                   jax.ShapeDtypeStruct((nv, L, bs), jnp.float32)],
        scratch_shapes=[pltpu.VMEM((L, bs), jnp.int32),
                        pltpu.VMEM((L, bs), jnp.float32),
                        pltpu.VMEM((L, bs), jnp.float32)],
        compiler_params=pltpu.CompilerParams(
            dimension_semantics=("arbitrary",)),
    )(xt)
    return jnp.transpose(xm_t, (3, 0, 1, 2)), jnp.transpose(mask_t, (2, 0, 1))
